# baseline (speedup 1.0000x reference)
import jax
import jax.numpy as jnp
from jax.experimental import pallas as pl
from jax.experimental.pallas import tpu as pltpu

N, NFEAT, NHID, DHID1 = 4096, 128, 64, 32
H = N // 2
BM_ENC = 512
NE = H // BM_ENC
BM_DEC = 512
ND = N // BM_DEC


def _encode(adj_blk, xw, bgc, w1, b1, w2, b2, w3, b3, wdec, z_scr, zw_scr,
            row0):
    h = jnp.dot(adj_blk, xw, preferred_element_type=jnp.float32)
    h = jnp.maximum(h + bgc, 0.0)
    h = jnp.maximum(jnp.dot(h, w1, preferred_element_type=jnp.float32) + b1,
                    0.0)
    h = jnp.maximum(jnp.dot(h, w2, preferred_element_type=jnp.float32) + b2,
                    0.0)
    z = jnp.dot(h, w3, preferred_element_type=jnp.float32) + b3
    z_scr[pl.ds(row0, BM_ENC), :] = z
    zw_scr[pl.ds(row0, BM_ENC), :] = jnp.dot(
        z, wdec, preferred_element_type=jnp.float32)


def _body(top_ref, bot_ref, x_ref, wgc_ref, bgc_ref, w1_ref, b1_ref, w2_ref,
          b2_ref, w3_ref, b3_ref, wdec_ref, out_ref, xw_scr, z_scr, zw_scr):
    i = pl.program_id(0)

    @pl.when(i == 0)
    def _():
        xw_scr[...] = jnp.dot(x_ref[...], wgc_ref[...],
                              preferred_element_type=jnp.float32)

    @pl.when(i < NE)
    def _():
        xw = xw_scr[...]
        args = (bgc_ref[...], w1_ref[...], b1_ref[...], w2_ref[...],
                b2_ref[...], w3_ref[...], b3_ref[...], wdec_ref[...])
        _encode(top_ref[...], xw, *args, z_scr, zw_scr, i * BM_ENC)
        _encode(bot_ref[...], xw, *args, z_scr, zw_scr, H + i * BM_ENC)

    @pl.when(i >= NE)
    def _():
        j = i - NE
        out_ref[...] = jax.lax.dot_general(
            zw_scr[pl.ds(j * BM_DEC, BM_DEC), :], z_scr[...],
            (((1,), (1,)), ((), ())), preferred_element_type=jnp.float32)


@jax.jit
def kernel(x, adj_norm_pos, W_gc, b_gc, W1, b1, W2, b2, W3, b3, W_dec):
    full = lambda shape: pl.BlockSpec(shape, lambda i: (0,) * len(shape))
    pin = lambda off: (lambda i: (jax.lax.min(i, NE - 1) + off, 0))

    logits = pl.pallas_call(
        _body,
        grid=(NE + ND,),
        in_specs=[
            pl.BlockSpec((BM_ENC, N), pin(0)),
            pl.BlockSpec((BM_ENC, N), pin(H // BM_ENC)),
            full((N, NFEAT)),
            full((NFEAT, NHID)),
            full((1, NHID)),
            full((NHID, DHID1)),
            full((1, DHID1)),
            full((DHID1, 2 * DHID1)),
            full((1, 2 * DHID1)),
            full((2 * DHID1, DHID1)),
            full((1, DHID1)),
            full((DHID1, DHID1)),
        ],
        out_specs=pl.BlockSpec((BM_DEC, N),
                               lambda i: (jax.lax.max(i - NE, 0), 0)),
        out_shape=jax.ShapeDtypeStruct((N, N), jnp.float32),
        scratch_shapes=[
            pltpu.VMEM((N, NHID), jnp.float32),
            pltpu.VMEM((N, DHID1), jnp.float32),
            pltpu.VMEM((N, DHID1), jnp.float32),
        ],
        compiler_params=pltpu.CompilerParams(
            dimension_semantics=("arbitrary",)),
    )(adj_norm_pos, adj_norm_pos, x, W_gc, b_gc.reshape(1, -1),
      W1, b1.reshape(1, -1), W2, b2.reshape(1, -1), W3, b3.reshape(1, -1),
      W_dec)
    return logits


# speedup vs baseline: 1.0790x; 1.0790x over previous
import jax
import jax.numpy as jnp
from jax.experimental import pallas as pl
from jax.experimental.pallas import tpu as pltpu

N, NFEAT, NHID, DHID1 = 4096, 128, 64, 32
BMR = 512
NE = N // BMR
NBUF = 3
BM_DEC = 512
ND = N // BM_DEC


def _body(adj_hbm, x_ref, wgc_ref, bgc_ref, w1_ref, b1_ref, w2_ref,
          b2_ref, w3_ref, b3_ref, wdec_ref, out_ref, abuf, sems, xw_scr,
          z_scr, zw_scr):
    i = pl.program_id(0)

    @pl.when(i == 0)
    def _():
        for b in range(NBUF):
            pltpu.make_async_copy(
                adj_hbm.at[pl.ds(b * BMR, BMR), :], abuf.at[b], sems.at[b]
            ).start()
        xw_scr[...] = jnp.dot(x_ref[...], wgc_ref[...],
                              preferred_element_type=jnp.float32)

    @pl.when(i < NE)
    def _():
        slot = jax.lax.rem(i, NBUF)
        pltpu.make_async_copy(
            adj_hbm.at[pl.ds(i * BMR, BMR), :], abuf.at[slot],
            sems.at[slot]).wait()
        h = jnp.dot(abuf[slot], xw_scr[...],
                    preferred_element_type=jnp.float32)
        h = jnp.maximum(h + bgc_ref[...], 0.0)
        h = jnp.maximum(jnp.dot(h, w1_ref[...],
                                preferred_element_type=jnp.float32)
                        + b1_ref[...], 0.0)
        h = jnp.maximum(jnp.dot(h, w2_ref[...],
                                preferred_element_type=jnp.float32)
                        + b2_ref[...], 0.0)
        z = (jnp.dot(h, w3_ref[...], preferred_element_type=jnp.float32)
             + b3_ref[...])
        z_scr[pl.ds(i * BMR, BMR), :] = z
        zw_scr[pl.ds(i * BMR, BMR), :] = jnp.dot(
            z, wdec_ref[...], preferred_element_type=jnp.float32)

        @pl.when(i + NBUF < NE)
        def _():
            pltpu.make_async_copy(
                adj_hbm.at[pl.ds((i + NBUF) * BMR, BMR), :],
                abuf.at[slot], sems.at[slot]).start()

    @pl.when(i >= NE)
    def _():
        j = i - NE
        out_ref[...] = jax.lax.dot_general(
            zw_scr[pl.ds(j * BM_DEC, BM_DEC), :], z_scr[...],
            (((1,), (1,)), ((), ())), preferred_element_type=jnp.float32)


@jax.jit
def kernel(x, adj_norm_pos, W_gc, b_gc, W1, b1, W2, b2, W3, b3, W_dec):
    full = lambda shape: pl.BlockSpec(shape, lambda i: (0,) * len(shape))

    logits = pl.pallas_call(
        _body,
        grid=(NE + ND,),
        in_specs=[
            pl.BlockSpec(memory_space=pl.ANY),
            full((N, NFEAT)),
            full((NFEAT, NHID)),
            full((1, NHID)),
            full((NHID, DHID1)),
            full((1, DHID1)),
            full((DHID1, 2 * DHID1)),
            full((1, 2 * DHID1)),
            full((2 * DHID1, DHID1)),
            full((1, DHID1)),
            full((DHID1, DHID1)),
        ],
        out_specs=pl.BlockSpec((BM_DEC, N),
                               lambda i: (jax.lax.max(i - NE, 0), 0)),
        out_shape=jax.ShapeDtypeStruct((N, N), jnp.float32),
        scratch_shapes=[
            pltpu.VMEM((NBUF, BMR, N), jnp.float32),
            pltpu.SemaphoreType.DMA((NBUF,)),
            pltpu.VMEM((N, NHID), jnp.float32),
            pltpu.VMEM((N, DHID1), jnp.float32),
            pltpu.VMEM((N, DHID1), jnp.float32),
        ],
        compiler_params=pltpu.CompilerParams(
            dimension_semantics=("arbitrary",)),
    )(adj_norm_pos, x, W_gc, b_gc.reshape(1, -1), W1, b1.reshape(1, -1),
      W2, b2.reshape(1, -1), W3, b3.reshape(1, -1), W_dec)
    return logits


# speedup vs baseline: 1.0861x; 1.0066x over previous
import jax
import jax.numpy as jnp
from jax.experimental import pallas as pl
from jax.experimental.pallas import tpu as pltpu

N, NFEAT, NHID, DHID1 = 4096, 128, 64, 32
BMR = 512
NE = N // BMR
NBUF = 4
BM_DEC = 512
ND = N // BM_DEC


def _body(adj_hbm, x_ref, wgc_ref, bgc_ref, w1_ref, b1_ref, w2_ref,
          b2_ref, w3_ref, b3_ref, wdec_ref, out_ref, abuf, sems, xw_scr,
          z_scr, zw_scr):
    i = pl.program_id(0)

    @pl.when(i == 0)
    def _():
        for b in range(NBUF - 1):
            pltpu.make_async_copy(
                adj_hbm.at[pl.ds(b * BMR, BMR), :], abuf.at[b], sems.at[b]
            ).start()
        xw_scr[...] = jnp.dot(x_ref[...], wgc_ref[...],
                              preferred_element_type=jnp.float32)

    @pl.when(i < NE)
    def _():
        slot = jax.lax.rem(i, NBUF)
        pltpu.make_async_copy(
            adj_hbm.at[pl.ds(i * BMR, BMR), :], abuf.at[slot],
            sems.at[slot]).wait()

        @pl.when(i + NBUF - 1 < NE)
        def _():
            nslot = jax.lax.rem(i + NBUF - 1, NBUF)
            pltpu.make_async_copy(
                adj_hbm.at[pl.ds((i + NBUF - 1) * BMR, BMR), :],
                abuf.at[nslot], sems.at[nslot]).start()

        h = jnp.dot(abuf[slot], xw_scr[...],
                    preferred_element_type=jnp.float32)
        h = jnp.maximum(h + bgc_ref[...], 0.0)
        h = jnp.maximum(jnp.dot(h, w1_ref[...],
                                preferred_element_type=jnp.float32)
                        + b1_ref[...], 0.0)
        h = jnp.maximum(jnp.dot(h, w2_ref[...],
                                preferred_element_type=jnp.float32)
                        + b2_ref[...], 0.0)
        z = (jnp.dot(h, w3_ref[...], preferred_element_type=jnp.float32)
             + b3_ref[...])
        z_scr[pl.ds(i * BMR, BMR), :] = z
        zw_scr[pl.ds(i * BMR, BMR), :] = jnp.dot(
            z, wdec_ref[...], preferred_element_type=jnp.float32)

    @pl.when(i >= NE)
    def _():
        j = i - NE
        out_ref[...] = jax.lax.dot_general(
            zw_scr[pl.ds(j * BM_DEC, BM_DEC), :], z_scr[...],
            (((1,), (1,)), ((), ())), preferred_element_type=jnp.float32)


@jax.jit
def kernel(x, adj_norm_pos, W_gc, b_gc, W1, b1, W2, b2, W3, b3, W_dec):
    full = lambda shape: pl.BlockSpec(shape, lambda i: (0,) * len(shape))

    logits = pl.pallas_call(
        _body,
        grid=(NE + ND,),
        in_specs=[
            pl.BlockSpec(memory_space=pl.ANY),
            full((N, NFEAT)),
            full((NFEAT, NHID)),
            full((1, NHID)),
            full((NHID, DHID1)),
            full((1, DHID1)),
            full((DHID1, 2 * DHID1)),
            full((1, 2 * DHID1)),
            full((2 * DHID1, DHID1)),
            full((1, DHID1)),
            full((DHID1, DHID1)),
        ],
        out_specs=pl.BlockSpec((BM_DEC, N),
                               lambda i: (jax.lax.max(i - NE, 0), 0)),
        out_shape=jax.ShapeDtypeStruct((N, N), jnp.float32),
        scratch_shapes=[
            pltpu.VMEM((NBUF, BMR, N), jnp.float32),
            pltpu.SemaphoreType.DMA((NBUF,)),
            pltpu.VMEM((N, NHID), jnp.float32),
            pltpu.VMEM((N, DHID1), jnp.float32),
            pltpu.VMEM((N, DHID1), jnp.float32),
        ],
        compiler_params=pltpu.CompilerParams(
            dimension_semantics=("arbitrary",)),
    )(adj_norm_pos, x, W_gc, b_gc.reshape(1, -1), W1, b1.reshape(1, -1),
      W2, b2.reshape(1, -1), W3, b3.reshape(1, -1), W_dec)
    return logits
